# Initial kernel scaffold; baseline (speedup 1.0000x reference)
import jax
import jax.numpy as jnp
from jax.experimental import pallas as pl


def kernel(x, edge_index, W, b, alpha):
    raise NotImplementedError("write your pallas kernel here")


# speedup vs baseline: 16.5492x; 16.5492x over previous
import functools

import jax
import jax.numpy as jnp
from jax import lax
from jax.experimental import pallas as pl
from jax.experimental.pallas import tpu as pltpu
from jax.experimental.pallas import tpu_sc as plsc

NC = 2
NS = 16
NW = NC * NS
CC = 128


def _deg_call(npad, rows_per_w):
    sl = npad // NS

    mesh = plsc.VectorSubcoreMesh(core_axis_name="c", subcore_axis_name="s")

    @functools.partial(
        pl.kernel,
        mesh=mesh,
        out_type=jax.ShapeDtypeStruct((NC * npad,), jnp.float32),
        scratch_types=[
            pltpu.VMEM((rows_per_w, CC), jnp.int32),
            pltpu.VMEM((CC,), jnp.float32),
            pltpu.VMEM((sl,), jnp.float32),
            pltpu.VMEM_SHARED((npad,), jnp.float32),
        ],
    )
    def deg_kernel(dst_hbm, deg_out, dstb, onesb, zb, deg_sh):
        cid = lax.axis_index("c")
        sid = lax.axis_index("s")
        z16 = jnp.zeros((16,), jnp.float32)
        o16 = jnp.ones((16,), jnp.float32)

        def zf(i, c):
            zb[pl.ds(i * 16, 16)] = z16
            return c
        lax.fori_loop(0, sl // 16, zf, 0)
        for k in range(CC // 16):
            onesb[pl.ds(k * 16, 16)] = o16

        base = sid * sl
        pltpu.sync_copy(zb, deg_sh.at[pl.ds(base, sl)])
        plsc.subcore_barrier()

        wid = cid * NS + sid
        pltpu.sync_copy(dst_hbm.at[pl.ds(wid * rows_per_w, rows_per_w)], dstb)

        def body(j, c):
            pltpu.sync_copy(onesb, deg_sh.at[dstb.at[j]], add=True)
            return c
        lax.fori_loop(0, rows_per_w, body, 0)
        plsc.subcore_barrier()
        pltpu.sync_copy(deg_sh.at[pl.ds(base, sl)],
                        deg_out.at[pl.ds(cid * npad + base, sl)])

    return deg_kernel


def _agg_call(npad, dh, rows_per_w):
    sl = npad // NS

    mesh = plsc.VectorSubcoreMesh(core_axis_name="c", subcore_axis_name="s")

    @functools.partial(
        pl.kernel,
        mesh=mesh,
        out_type=jax.ShapeDtypeStruct((NC, 2, npad, dh), jnp.float32),
        compiler_params=pltpu.CompilerParams(use_tc_tiling_on_sc=False),
        scratch_types=[
            pltpu.VMEM((rows_per_w, CC), jnp.int32),
            pltpu.VMEM((rows_per_w, CC), jnp.int32),
            pltpu.VMEM((CC, dh), jnp.float32),
            pltpu.VMEM((CC, dh), jnp.float32),
            pltpu.VMEM((CC, dh), jnp.float32),
            pltpu.VMEM_SHARED((npad, dh), jnp.float32),
            pltpu.SemaphoreType.DMA,
            pltpu.SemaphoreType.DMA,
        ],
    )
    def agg_kernel(xs0_hbm, xs1_hbm, src_hbm, dst_hbm, agg_out,
                   srcb, dstb, rba, rbb, zb, agg_sh, sga, sgb):
        cid = lax.axis_index("c")
        sid = lax.axis_index("s")
        z16 = jnp.zeros((16,), jnp.float32)

        def zf(i, c):
            r = i // (dh // 16)
            k = i % (dh // 16)
            zb[r, pl.ds(k * 16, 16)] = z16
            return c
        lax.fori_loop(0, CC * (dh // 16), zf, 0)

        base = sid * sl
        wid = cid * NS + sid
        wrow = wid * rows_per_w
        pltpu.sync_copy(src_hbm.at[pl.ds(wrow, rows_per_w)], srcb)
        pltpu.sync_copy(dst_hbm.at[pl.ds(wrow, rows_per_w)], dstb)

        for f, xs_hbm in ((0, xs0_hbm), (1, xs1_hbm)):
            for k in range(sl // CC):
                pltpu.sync_copy(zb, agg_sh.at[pl.ds(base + k * CC, CC), :])
            plsc.subcore_barrier()

            def g_start(j, buf, sem):
                pltpu.make_async_copy(xs_hbm.at[srcb.at[j]], buf, sem).start()

            def g_wait(j, buf, sem):
                pltpu.make_async_copy(xs_hbm.at[srcb.at[j]], buf, sem).wait()

            def s_sync(j, buf):
                pltpu.sync_copy(buf, agg_sh.at[dstb.at[j]], add=True)

            g_start(0, rba, sga)

            def body(g, c):
                j = g * 2
                g_wait(j, rba, sga)

                @pl.when(j + 1 < rows_per_w)
                def _():
                    g_start(j + 1, rbb, sgb)
                s_sync(j, rba)

                @pl.when(j + 1 < rows_per_w)
                def _():
                    g_wait(j + 1, rbb, sgb)

                    @pl.when(j + 2 < rows_per_w)
                    def _():
                        g_start(j + 2, rba, sga)
                    s_sync(j + 1, rbb)
                return c
            lax.fori_loop(0, (rows_per_w + 1) // 2, body, 0)

            plsc.subcore_barrier()
            for k in range(sl // CC):
                pltpu.sync_copy(agg_sh.at[pl.ds(base + k * CC, CC), :],
                                agg_out.at[cid, f, pl.ds(base + k * CC, CC), :])
            plsc.subcore_barrier()

    return agg_kernel


def _prep_kernel(degp_ref, x_ref, dinv_ref, xsh_ref):
    deg = degp_ref[0] + degp_ref[1] + 1.0
    dinv = lax.rsqrt(deg)
    dinv_ref[...] = dinv
    xs = x_ref[...] * dinv
    dh = xs.shape[1] // 2
    xsh_ref[0] = xs[:, :dh]
    xsh_ref[1] = xs[:, dh:]


def _out_kernel(agg_ref, xsh_ref, dinv_ref, w_ref, b_ref, a_ref, o_ref):
    dinv = dinv_ref[...]
    dh = xsh_ref.shape[2]
    pre0 = dinv * (agg_ref[0, 0] + agg_ref[1, 0] + xsh_ref[0])
    pre1 = dinv * (agg_ref[0, 1] + agg_ref[1, 1] + xsh_ref[1])
    w = w_ref[...]
    h = (jnp.dot(pre0, w[:dh], preferred_element_type=jnp.float32)
         + jnp.dot(pre1, w[dh:], preferred_element_type=jnp.float32))
    h = h + b_ref[...]
    o_ref[...] = jnp.where(h >= 0.0, h, a_ref[...] * h)


def kernel(x, edge_index, W, b, alpha):
    n, din = x.shape
    e = edge_index.shape[1]
    dout = W.shape[1]
    assert din % 16 == 0
    npad = ((n + NS * 16 - 1) // (NS * 16)) * (NS * 16)
    assert npad > n
    epad = -(-e // (CC * NW * 8)) * (CC * NW * 8)
    rows = epad // CC
    rows_per_w = rows // NW

    pad_e = jnp.full((epad - e,), npad - 1, jnp.int32)
    src2 = jnp.concatenate([edge_index[0], pad_e]).reshape(rows, CC)
    dst2 = jnp.concatenate([edge_index[1], pad_e]).reshape(rows, CC)
    xpad = jnp.concatenate([x, jnp.zeros((npad - n, din), x.dtype)], axis=0)

    degp = _deg_call(npad, rows_per_w)(dst2)
    degp3 = degp.reshape(NC, npad, 1)

    dh = din // 2
    dinv, xsh = pl.pallas_call(
        _prep_kernel,
        out_shape=[
            jax.ShapeDtypeStruct((npad, 1), jnp.float32),
            jax.ShapeDtypeStruct((2, npad, dh), jnp.float32),
        ],
    )(degp3, xpad)

    aggp = _agg_call(npad, dh, rows_per_w)(xsh[0], xsh[1], src2, dst2)

    blk = 640
    grid = ((n + blk - 1) // blk,)
    out = pl.pallas_call(
        _out_kernel,
        grid=grid,
        in_specs=[
            pl.BlockSpec((NC, 2, blk, dh), lambda i: (0, 0, i, 0)),
            pl.BlockSpec((2, blk, dh), lambda i: (0, i, 0)),
            pl.BlockSpec((blk, 1), lambda i: (i, 0)),
            pl.BlockSpec((din, dout), lambda i: (0, 0)),
            pl.BlockSpec((1, dout), lambda i: (0, 0)),
            pl.BlockSpec((1, dout), lambda i: (0, 0)),
        ],
        out_specs=pl.BlockSpec((blk, dout), lambda i: (i, 0)),
        out_shape=jax.ShapeDtypeStruct((n, dout), jnp.float32),
    )(aggp, xsh, dinv, W, b.reshape(1, dout), alpha.reshape(1, dout))
    return out


# speedup vs baseline: 17.8954x; 1.0813x over previous
import functools

import jax
import jax.numpy as jnp
from jax import lax
from jax.experimental import pallas as pl
from jax.experimental.pallas import tpu as pltpu
from jax.experimental.pallas import tpu_sc as plsc

NC = 2
NS = 16
NW = NC * NS
CC = 128


def _deg_call(npad, rows_per_w):
    sl = npad // NS

    mesh = plsc.VectorSubcoreMesh(core_axis_name="c", subcore_axis_name="s")

    @functools.partial(
        pl.kernel,
        mesh=mesh,
        out_type=jax.ShapeDtypeStruct((NC * npad,), jnp.float32),
        scratch_types=[
            pltpu.VMEM((rows_per_w, CC), jnp.int32),
            pltpu.VMEM((CC,), jnp.float32),
            pltpu.VMEM((sl,), jnp.float32),
            pltpu.VMEM_SHARED((npad,), jnp.float32),
        ],
    )
    def deg_kernel(dst_hbm, deg_out, dstb, onesb, zb, deg_sh):
        cid = lax.axis_index("c")
        sid = lax.axis_index("s")
        z16 = jnp.zeros((16,), jnp.float32)
        o16 = jnp.ones((16,), jnp.float32)

        def zf(i, c):
            zb[pl.ds(i * 16, 16)] = z16
            return c
        lax.fori_loop(0, sl // 16, zf, 0)
        for k in range(CC // 16):
            onesb[pl.ds(k * 16, 16)] = o16

        base = sid * sl
        pltpu.sync_copy(zb, deg_sh.at[pl.ds(base, sl)])
        plsc.subcore_barrier()

        wid = cid * NS + sid
        pltpu.sync_copy(dst_hbm.at[pl.ds(wid * rows_per_w, rows_per_w)], dstb)

        def body(j, c):
            pltpu.sync_copy(onesb, deg_sh.at[dstb.at[j]], add=True)
            return c
        lax.fori_loop(0, rows_per_w, body, 0)
        plsc.subcore_barrier()
        pltpu.sync_copy(deg_sh.at[pl.ds(base, sl)],
                        deg_out.at[pl.ds(cid * npad + base, sl)])

    return deg_kernel


def _agg_call(npad, dh, rows_per_w):
    sl = npad // NS

    mesh = plsc.VectorSubcoreMesh(core_axis_name="c", subcore_axis_name="s")

    @functools.partial(
        pl.kernel,
        mesh=mesh,
        out_type=jax.ShapeDtypeStruct((NC, 2, npad, dh), jnp.float32),
        compiler_params=pltpu.CompilerParams(use_tc_tiling_on_sc=False),
        scratch_types=[
            pltpu.VMEM((rows_per_w, CC), jnp.int32),
            pltpu.VMEM((rows_per_w, CC), jnp.int32),
            pltpu.VMEM_SHARED((npad, dh), jnp.float32),
        ] + [pltpu.VMEM((CC, dh), jnp.float32) for _ in range(8)]
          + [pltpu.SemaphoreType.DMA for _ in range(16)],
    )
    def agg_kernel(xs0_hbm, xs1_hbm, src_hbm, dst_hbm, agg_out,
                   srcb, dstb, agg_sh, *ring):
        bufs, sg, ss = ring[0:8], ring[8:16], ring[16:24]
        cid = lax.axis_index("c")
        sid = lax.axis_index("s")
        z16 = jnp.zeros((16,), jnp.float32)

        def zf(i, c):
            r = i // (dh // 16)
            k = i % (dh // 16)
            bufs[0][r, pl.ds(k * 16, 16)] = z16
            return c

        base = sid * sl
        wid = cid * NS + sid
        wrow = wid * rows_per_w
        pltpu.sync_copy(src_hbm.at[pl.ds(wrow, rows_per_w)], srcb)
        pltpu.sync_copy(dst_hbm.at[pl.ds(wrow, rows_per_w)], dstb)

        for f, xs_hbm in ((0, xs0_hbm), (1, xs1_hbm)):
            lax.fori_loop(0, CC * (dh // 16), zf, 0)
            for k in range(sl // CC):
                pltpu.sync_copy(bufs[0], agg_sh.at[pl.ds(base + k * CC, CC), :])
            plsc.subcore_barrier()

            def g_start(j, b):
                pltpu.make_async_copy(
                    xs_hbm.at[srcb.at[j]], bufs[b], sg[b]).start()

            def g_wait(j, b):
                pltpu.make_async_copy(
                    xs_hbm.at[srcb.at[j]], bufs[b], sg[b]).wait()

            def s_start(j, b):
                pltpu.async_copy(
                    bufs[b], agg_sh.at[dstb.at[j]], ss[b], add=True)

            def s_wait(j, b):
                pltpu.make_async_copy(
                    bufs[b], agg_sh.at[dstb.at[j]], ss[b]).wait()

            for b in range(4):
                g_start(b, b)

            def body(g, c):
                for b in range(8):
                    j = g * 8 + b
                    bp = (b + 4) % 8
                    g_wait(j, b)
                    s_start(j, b)
                    nxt = j + 4

                    @pl.when(nxt < rows_per_w)
                    def _():
                        @pl.when(j >= 4)
                        def _():
                            s_wait(j - 4, bp)
                        g_start(nxt, bp)
                return c
            lax.fori_loop(0, rows_per_w // 8, body, 0)
            for b in range(8):
                s_wait(rows_per_w - 8 + b, b)

            plsc.subcore_barrier()
            for k in range(sl // CC):
                pltpu.sync_copy(agg_sh.at[pl.ds(base + k * CC, CC), :],
                                agg_out.at[cid, f, pl.ds(base + k * CC, CC), :])
            plsc.subcore_barrier()

    return agg_kernel


def _prep_kernel(degp_ref, x_ref, dinv_ref, xsh_ref):
    deg = degp_ref[0] + degp_ref[1] + 1.0
    dinv = lax.rsqrt(deg)
    dinv_ref[...] = dinv
    xs = x_ref[...] * dinv
    dh = xs.shape[1] // 2
    xsh_ref[0] = xs[:, :dh]
    xsh_ref[1] = xs[:, dh:]


def _out_kernel(agg_ref, xsh_ref, dinv_ref, w_ref, b_ref, a_ref, o_ref):
    dinv = dinv_ref[...]
    dh = xsh_ref.shape[2]
    pre0 = dinv * (agg_ref[0, 0] + agg_ref[1, 0] + xsh_ref[0])
    pre1 = dinv * (agg_ref[0, 1] + agg_ref[1, 1] + xsh_ref[1])
    w = w_ref[...]
    h = (jnp.dot(pre0, w[:dh], preferred_element_type=jnp.float32)
         + jnp.dot(pre1, w[dh:], preferred_element_type=jnp.float32))
    h = h + b_ref[...]
    o_ref[...] = jnp.where(h >= 0.0, h, a_ref[...] * h)


def kernel(x, edge_index, W, b, alpha):
    n, din = x.shape
    e = edge_index.shape[1]
    dout = W.shape[1]
    assert din % 16 == 0
    npad = ((n + NS * 16 - 1) // (NS * 16)) * (NS * 16)
    assert npad > n
    epad = -(-e // (CC * NW * 8)) * (CC * NW * 8)
    rows = epad // CC
    rows_per_w = rows // NW

    pad_e = jnp.full((epad - e,), npad - 1, jnp.int32)
    src2 = jnp.concatenate([edge_index[0], pad_e]).reshape(rows, CC)
    dst2 = jnp.concatenate([edge_index[1], pad_e]).reshape(rows, CC)
    xpad = jnp.concatenate([x, jnp.zeros((npad - n, din), x.dtype)], axis=0)

    degp = _deg_call(npad, rows_per_w)(dst2)
    degp3 = degp.reshape(NC, npad, 1)

    dh = din // 2
    dinv, xsh = pl.pallas_call(
        _prep_kernel,
        out_shape=[
            jax.ShapeDtypeStruct((npad, 1), jnp.float32),
            jax.ShapeDtypeStruct((2, npad, dh), jnp.float32),
        ],
    )(degp3, xpad)

    aggp = _agg_call(npad, dh, rows_per_w)(xsh[0], xsh[1], src2, dst2)

    blk = 640
    grid = ((n + blk - 1) // blk,)
    out = pl.pallas_call(
        _out_kernel,
        grid=grid,
        in_specs=[
            pl.BlockSpec((NC, 2, blk, dh), lambda i: (0, 0, i, 0)),
            pl.BlockSpec((2, blk, dh), lambda i: (0, i, 0)),
            pl.BlockSpec((blk, 1), lambda i: (i, 0)),
            pl.BlockSpec((din, dout), lambda i: (0, 0)),
            pl.BlockSpec((1, dout), lambda i: (0, 0)),
            pl.BlockSpec((1, dout), lambda i: (0, 0)),
        ],
        out_specs=pl.BlockSpec((blk, dout), lambda i: (i, 0)),
        out_shape=jax.ShapeDtypeStruct((n, dout), jnp.float32),
    )(aggp, xsh, dinv, W, b.reshape(1, dout), alpha.reshape(1, dout))
    return out
